# baseline (speedup 1.0000x reference)
import functools

import jax
import jax.numpy as jnp
from jax import lax
from jax.experimental import pallas as pl
from jax.experimental.pallas import tpu as pltpu
from jax.experimental.pallas import tpu_sc as plsc

NUM_WORKERS = 32
CHUNK = 128
NBUF = 8
AHEAD = 4


def _make_gather(n_ids: int, vocab: int, padded_dim: int):
  n_per_w = n_ids // NUM_WORKERS
  n_chunks = n_per_w // CHUNK
  mesh = plsc.VectorSubcoreMesh(core_axis_name="c", subcore_axis_name="s")

  dim = 64

  @functools.partial(
      pl.kernel,
      mesh=mesh,
      out_type=jax.ShapeDtypeStruct((n_ids, dim), jnp.float32),
      scratch_types=[
          pltpu.VMEM((n_chunks, CHUNK), jnp.int32),
          pltpu.VMEM((NBUF, CHUNK, dim), jnp.float32),
          pltpu.SemaphoreType.DMA((NBUF,)),
          pltpu.SemaphoreType.DMA((NBUF,)),
      ],
      compiler_params=pltpu.CompilerParams(use_tc_tiling_on_sc=False,
                                           needs_layout_passes=False),
  )
  def gather_kernel(ids_hbm, table_hbm, out_hbm, idx_v, rows_v, gsem, wsem):
    c = lax.axis_index("c")
    s = lax.axis_index("s")
    wid = s * 2 + c
    base = wid * n_per_w
    pltpu.sync_copy(ids_hbm.at[wid], idx_v)

    def gather(j, slot):
      pltpu.async_copy(table_hbm.at[idx_v.at[j]], rows_v.at[slot],
                       gsem.at[slot])

    def wait_gather(j, slot):
      pltpu.make_async_copy(table_hbm.at[idx_v.at[j]], rows_v.at[slot],
                            gsem.at[slot]).wait()

    def put(j, slot):
      pltpu.async_copy(rows_v.at[slot],
                       out_hbm.at[pl.ds(base + j * CHUNK, CHUNK)],
                       wsem.at[slot])

    def wait_put(j, slot):
      pltpu.make_async_copy(rows_v.at[slot],
                            out_hbm.at[pl.ds(base + j * CHUNK, CHUNK)],
                            wsem.at[slot]).wait()

    for b in range(AHEAD):
      gather(b, b)

    for j in range(NBUF - AHEAD):
      wait_gather(j, j)
      put(j, j)
      gather(j + AHEAD, (j + AHEAD) % NBUF)

    def body(j, carry):
      slot = lax.rem(j, NBUF)
      wait_gather(j, slot)
      put(j, slot)
      nslot = lax.rem(j + AHEAD, NBUF)
      wait_put(j + AHEAD - NBUF, nslot)
      gather(j + AHEAD, nslot)
      return carry

    lax.fori_loop(NBUF - AHEAD, n_chunks - AHEAD, body, 0)

    for k in range(AHEAD):
      j = n_chunks - AHEAD + k
      slot = j % NBUF
      wait_gather(j, slot)
      put(j, slot)
    for k in range(NBUF):
      j = n_chunks - NBUF + k
      wait_put(j, j % NBUF)

  return gather_kernel


def kernel(input_ids, attention_mask, emb_weight):
  batch, seq = input_ids.shape
  vocab, dim = emb_weight.shape
  n_ids = batch * seq
  table_wide = jax.lax.optimization_barrier(emb_weight.reshape(vocab // 2, 2 * dim))
  table = table_wide.reshape(vocab, dim)
  ids = input_ids.reshape(NUM_WORKERS, n_ids // (NUM_WORKERS * CHUNK), CHUNK)
  ids = ids.astype(jnp.int32)
  out = _make_gather(n_ids, vocab, 128)(ids, table)
  return out.reshape(batch, seq, dim), attention_mask


# speedup vs baseline: 1.3270x; 1.3270x over previous
import functools

import jax
import jax.numpy as jnp
from jax import lax
from jax.experimental import pallas as pl
from jax.experimental.pallas import tpu as pltpu
from jax.experimental.pallas import tpu_sc as plsc

NUM_WORKERS = 32
CHUNK = 128
NBUF = 8
AHEAD = 4


def _make_gather(n_ids: int, vocab: int, padded_dim: int):
  n_per_w = n_ids // NUM_WORKERS
  n_chunks = n_per_w // CHUNK
  mesh = plsc.VectorSubcoreMesh(core_axis_name="c", subcore_axis_name="s")

  dim = 64

  @functools.partial(
      pl.kernel,
      mesh=mesh,
      out_type=jax.ShapeDtypeStruct((n_ids, padded_dim), jnp.float32),
      scratch_types=[
          pltpu.VMEM((n_chunks, CHUNK), jnp.int32),
          pltpu.VMEM((NBUF, CHUNK, dim), jnp.float32),
          pltpu.SemaphoreType.DMA((NBUF,)),
          pltpu.SemaphoreType.DMA((NBUF,)),
      ],
      compiler_params=pltpu.CompilerParams(use_tc_tiling_on_sc=False,
                                           needs_layout_passes=False),
  )
  def gather_kernel(ids_hbm, table_hbm, out_hbm, idx_v, rows_v, gsem, wsem):
    c = lax.axis_index("c")
    s = lax.axis_index("s")
    wid = s * 2 + c
    base = wid * n_per_w
    pltpu.sync_copy(ids_hbm.at[wid], idx_v)

    def gather(j, slot):
      pltpu.async_copy(table_hbm.at[idx_v.at[j]], rows_v.at[slot],
                       gsem.at[slot])

    def wait_gather(j, slot):
      pltpu.make_async_copy(table_hbm.at[idx_v.at[j]], rows_v.at[slot],
                            gsem.at[slot]).wait()

    def put(j, slot):
      pltpu.async_copy(
          rows_v.at[slot],
          out_hbm.at[pl.ds(base + j * CHUNK, CHUNK), pl.ds(0, dim)],
          wsem.at[slot])

    def wait_put(j, slot):
      pltpu.make_async_copy(
          rows_v.at[slot],
          out_hbm.at[pl.ds(base + j * CHUNK, CHUNK), pl.ds(0, dim)],
          wsem.at[slot]).wait()

    for b in range(AHEAD):
      gather(b, b)

    for j in range(NBUF - AHEAD):
      wait_gather(j, j)
      put(j, j)
      gather(j + AHEAD, (j + AHEAD) % NBUF)

    def body(j, carry):
      slot = lax.rem(j, NBUF)
      wait_gather(j, slot)
      put(j, slot)
      nslot = lax.rem(j + AHEAD, NBUF)
      wait_put(j + AHEAD - NBUF, nslot)
      gather(j + AHEAD, nslot)
      return carry

    lax.fori_loop(NBUF - AHEAD, n_chunks - AHEAD, body, 0)

    for k in range(AHEAD):
      j = n_chunks - AHEAD + k
      slot = j % NBUF
      wait_gather(j, slot)
      put(j, slot)
    for k in range(NBUF):
      j = n_chunks - NBUF + k
      wait_put(j, j % NBUF)

  return gather_kernel


def kernel(input_ids, attention_mask, emb_weight):
  batch, seq = input_ids.shape
  vocab, dim = emb_weight.shape
  n_ids = batch * seq
  table_wide = jax.lax.optimization_barrier(emb_weight.reshape(vocab // 2, 2 * dim))
  table = table_wide.reshape(vocab, dim)
  ids = input_ids.reshape(NUM_WORKERS, n_ids // (NUM_WORKERS * CHUNK), CHUNK)
  ids = ids.astype(jnp.int32)
  out128 = _make_gather(n_ids, vocab, 128)(ids, table)
  return out128[:, :dim].reshape(batch, seq, dim), attention_mask
